# baseline (speedup 1.0000x reference)
import jax
import jax.numpy as jnp
from jax import lax
from jax.experimental import pallas as pl
from jax.experimental.pallas import tpu as pltpu
from jax.experimental.pallas import tpu_sc as plsc

N = 10000
E = 160000
F_IN = 256
D1 = 512
BN = 400
N_PAD = 10400
NBLK_PAD = N_PAD // BN
NBLK = N // BN
E_PAD = 163840
EPT = E_PAD // 32
CHUNK = 128
NCHUNK = EPT // CHUNK
STRIPE = N_PAD // 16
FULL_FLUSH = STRIPE // CHUNK
TAIL_FLUSH = STRIPE % CHUNK
ZROWS = 64
EROWS = E // CHUNK
EROWS_PAD = E_PAD // CHUNK
SENT = (N << 14) | N

_mesh = plsc.VectorSubcoreMesh(core_axis_name="c", subcore_axis_name="s")


def _leaky_exp(v):
    return jnp.exp(jnp.maximum(v, 0.2 * v))


def _tc_d_body(src_ref, dst_ref, enc_ref):
    enc = (src_ref[...] << 14) | dst_ref[...]
    pad = jnp.full((EROWS_PAD - EROWS, CHUNK), SENT, jnp.int32)
    enc_ref[...] = jnp.concatenate([enc, pad], axis=0)


def _tc_d(src2d, dst2d):
    return pl.pallas_call(
        _tc_d_body,
        out_shape=jax.ShapeDtypeStruct((EROWS_PAD, CHUNK), jnp.int32),
    )(src2d, dst2d)


def _tc_a_body(x_ref, w1_ref, a1_ref, *out_refs):
    h = jnp.dot(x_ref[...], w1_ref[...], preferred_element_type=jnp.float32)
    for g in range(8):
        out_refs[g][...] = h[:, g * 64:(g + 1) * 64]
    out_refs[8][...] = jnp.dot(h, a1_ref[...],
                               preferred_element_type=jnp.float32)


def _tc_a(x, W1, A1):
    return pl.pallas_call(
        _tc_a_body,
        grid=(NBLK_PAD,),
        in_specs=[
            pl.BlockSpec((BN, F_IN), lambda i: (jnp.minimum(i, NBLK - 1), 0)),
            pl.BlockSpec((F_IN, D1), lambda i: (0, 0)),
            pl.BlockSpec((D1, 16), lambda i: (0, 0)),
        ],
        out_specs=[pl.BlockSpec((BN, 64), lambda i: (i, 0))] * 8
        + [pl.BlockSpec((BN, 16), lambda i: (i, 0))],
        out_shape=[jax.ShapeDtypeStruct((N_PAD, 64), jnp.float32)] * 8
        + [jax.ShapeDtypeStruct((N_PAD, 16), jnp.float32)],
    )(x, W1, A1)


def _zero_stripe(zb, acc, row0):
    nfull = STRIPE // ZROWS
    for t in range(nfull):
        pltpu.sync_copy(zb, acc.at[pl.ds(row0 + t * ZROWS, ZROWS)])
    tail = STRIPE % ZROWS
    if tail:
        pltpu.sync_copy(zb.at[pl.ds(0, tail)],
                        acc.at[pl.ds(row0 + nfull * ZROWS, tail)])


def _flush_stripe(acc, buf, out_at, row0):
    for t in range(FULL_FLUSH):
        r = row0 + t * CHUNK
        pltpu.sync_copy(acc.at[pl.ds(r, CHUNK)], buf)
        pltpu.sync_copy(buf, out_at(r, CHUNK))
    r = row0 + FULL_FLUSH * CHUNK
    pltpu.sync_copy(acc.at[pl.ds(r, TAIL_FLUSH)], buf.at[pl.ds(0, TAIL_FLUSH)])
    pltpu.sync_copy(buf.at[pl.ds(0, TAIL_FLUSH)], out_at(r, TAIL_FLUSH))


def _sc1_body(enc_ref, t1_ref, h0_ref, h1_ref, h2_ref, h3_ref, h4_ref,
              h5_ref, h6_ref, h7_ref, num_ref, den_ref,
              ebuf, idx_s, idx_d, wstore, gbuf, tsb, tdb, zb, zbn, accum,
              dacc, sem):
    c = lax.axis_index("c")
    s = lax.axis_index("s")
    tid = c * 16 + s
    base = tid * EPT
    row0 = s * STRIPE
    shift8 = (lax.iota(jnp.int32, 16) + 8) % 16

    def zrow(i, carry):
        for j in range(4):
            zb[i, pl.ds(j * 16, 16)] = jnp.zeros((16,), jnp.float32)
        return carry

    lax.fori_loop(0, ZROWS, zrow, 0)

    htabs = [h0_ref, h1_ref, h2_ref, h3_ref, h4_ref, h5_ref, h6_ref, h7_ref]
    for cg in range(8):
        _zero_stripe(zb, accum, row0)
        if cg == 0:
            _zero_stripe(zbn, dacc, row0)
        plsc.subcore_barrier()

        def chunk_body(k, carry, cg=cg):
            off = pl.multiple_of(base + k * CHUNK, CHUNK)
            pltpu.sync_copy(enc_ref.at[pl.ds(off, CHUNK)], ebuf)
            for i in range(CHUNK // 16):
                ev = ebuf[pl.ds(i * 16, 16)]
                idx_s[pl.ds(i * 16, 16)] = ev >> 14
                idx_d[pl.ds(i * 16, 16)] = ev & 16383
            pltpu.async_copy(t1_ref.at[idx_s], tsb, sem).wait()
            pltpu.async_copy(t1_ref.at[idx_d], tdb, sem).wait()

            def wbody(e, carry2):
                u = tsb[e, :]
                v = tdb[e, :]
                wstore[e, :] = _leaky_exp(u + v[shift8])
                return carry2

            lax.fori_loop(0, CHUNK, wbody, 0)
            if cg == 0:
                pltpu.sync_copy(wstore, dacc.at[idx_d], add=True)
            pltpu.async_copy(htabs[cg].at[idx_s], gbuf, sem).wait()

            def mbody(e, carry2, cg=cg):
                w0 = wstore[e, :][cg]
                for j in range(4):
                    gbuf[e, pl.ds(j * 16, 16)] = gbuf[e, pl.ds(j * 16, 16)] * w0
                return carry2

            lax.fori_loop(0, CHUNK, mbody, 0)
            pltpu.sync_copy(gbuf, accum.at[idx_d], add=True)
            return carry

        lax.fori_loop(0, NCHUNK, chunk_body, 0)
        plsc.subcore_barrier()

        _flush_stripe(accum, gbuf,
                      lambda r, n, cg=cg: num_ref.at[c, cg, pl.ds(r, n)], row0)
        if cg == 0:
            _flush_stripe(dacc, tsb,
                          lambda r, n: den_ref.at[c, pl.ds(r, n)], row0)


def _sc1(enc_p, t1_p, htabs):
    f = pl.kernel(
        _sc1_body,
        out_type=[
            jax.ShapeDtypeStruct((2, 8, N_PAD, 64), jnp.float32),
            jax.ShapeDtypeStruct((2, N_PAD, 16), jnp.float32),
        ],
        mesh=_mesh,
        compiler_params=pltpu.CompilerParams(use_tc_tiling_on_sc=False),
        scratch_types=[
            pltpu.VMEM((CHUNK,), jnp.int32),
            pltpu.VMEM((CHUNK,), jnp.int32),
            pltpu.VMEM((CHUNK,), jnp.int32),
            pltpu.VMEM((CHUNK, 16), jnp.float32),
            pltpu.VMEM((CHUNK, 64), jnp.float32),
            pltpu.VMEM((CHUNK, 16), jnp.float32),
            pltpu.VMEM((CHUNK, 16), jnp.float32),
            pltpu.VMEM((ZROWS, 64), jnp.float32),
            pltpu.VMEM((ZROWS, 16), jnp.float32),
            pltpu.VMEM_SHARED((N_PAD, 64), jnp.float32),
            pltpu.VMEM_SHARED((N_PAD, 16), jnp.float32),
            pltpu.SemaphoreType.DMA,
        ],
    )
    return f(enc_p, t1_p, *htabs)


def _tc_b_body(num_ref, den_ref, t1_ref, h1g0, h1g1, h1g2, h1g3, h1g4,
               h1g5, h1g6, h1g7, b1_ref, w2_ref, a2_ref, h2_ref, t2_ref):
    hg = [h1g0, h1g1, h1g2, h1g3, h1g4, h1g5, h1g6, h1g7]
    nsum = num_ref[0] + num_ref[1]
    ncat = jnp.concatenate([nsum[g] for g in range(8)], axis=-1)
    hcat = jnp.concatenate([r[...] for r in hg], axis=-1)
    t1 = t1_ref[...]
    s1 = t1[:, 0:8] + t1[:, 8:16]
    w8 = _leaky_exp(s1)
    wc = jnp.concatenate(
        [jnp.broadcast_to(w8[:, h:h + 1], (BN, 64)) for h in range(8)],
        axis=-1)
    d8 = den_ref[0] + den_ref[1]
    dtot = d8[:, 0:8] + w8
    dc = jnp.concatenate(
        [jnp.broadcast_to(dtot[:, h:h + 1], (BN, 64)) for h in range(8)],
        axis=-1)
    numf = ncat + wc * hcat
    out1 = numf / (dc + 1e-16) + b1_ref[...]
    act = jnp.where(out1 > 0, out1,
                    jnp.exp(jnp.minimum(out1, 0.0)) - 1.0)
    h2 = jnp.dot(act, w2_ref[...], preferred_element_type=jnp.float32)
    ones = jnp.ones((BN, 8), jnp.float32)
    zeros = jnp.zeros((BN, 16), jnp.float32)
    h2_ref[...] = jnp.concatenate([h2[:, 0:40], ones, zeros], axis=-1)
    t2_ref[...] = jnp.dot(h2, a2_ref[...], preferred_element_type=jnp.float32)


def _tc_b(num1, den1, t1, h1g, b1, W2p, A2):
    return pl.pallas_call(
        _tc_b_body,
        grid=(NBLK_PAD,),
        in_specs=[
            pl.BlockSpec((2, 8, BN, 64), lambda i: (0, 0, i, 0)),
            pl.BlockSpec((2, BN, 16), lambda i: (0, i, 0)),
            pl.BlockSpec((BN, 16), lambda i: (i, 0)),
        ] + [pl.BlockSpec((BN, 64), lambda i: (i, 0))] * 8 + [
            pl.BlockSpec((D1,), lambda i: (0,)),
            pl.BlockSpec((D1, 64), lambda i: (0, 0)),
            pl.BlockSpec((64, 16), lambda i: (0, 0)),
        ],
        out_specs=[
            pl.BlockSpec((BN, 64), lambda i: (i, 0)),
            pl.BlockSpec((BN, 16), lambda i: (i, 0)),
        ],
        out_shape=[
            jax.ShapeDtypeStruct((N_PAD, 64), jnp.float32),
            jax.ShapeDtypeStruct((N_PAD, 16), jnp.float32),
        ],
    )(num1, den1, t1, *h1g, b1, W2p, A2)


def _sc2_body(enc_ref, t2_ref, h2_ref, num_ref,
              ebuf, idx_s, idx_d, gbuf, wrow, tsb, tdb, zb, accum, sem):
    c = lax.axis_index("c")
    s = lax.axis_index("s")
    tid = c * 16 + s
    base = tid * EPT
    row0 = s * STRIPE
    shift8 = (lax.iota(jnp.int32, 16) + 8) % 16
    low8 = lax.iota(jnp.int32, 16) % 8

    def zrow(i, carry):
        for j in range(4):
            zb[i, pl.ds(j * 16, 16)] = jnp.zeros((16,), jnp.float32)
        return carry

    lax.fori_loop(0, ZROWS, zrow, 0)

    _zero_stripe(zb, accum, row0)
    plsc.subcore_barrier()

    def chunk_body(k, carry):
        off = pl.multiple_of(base + k * CHUNK, CHUNK)
        pltpu.sync_copy(enc_ref.at[pl.ds(off, CHUNK)], ebuf)
        for i in range(CHUNK // 16):
            ev = ebuf[pl.ds(i * 16, 16)]
            idx_s[pl.ds(i * 16, 16)] = ev >> 14
            idx_d[pl.ds(i * 16, 16)] = ev & 16383
        pltpu.async_copy(t2_ref.at[idx_s], tsb, sem).wait()
        pltpu.async_copy(t2_ref.at[idx_d], tdb, sem).wait()

        def wbody(e, carry2):
            u = tsb[e, :]
            v = tdb[e, :]
            w = _leaky_exp(u + v[shift8])
            wrow[e, :] = w[low8]
            return carry2

        lax.fori_loop(0, CHUNK, wbody, 0)
        pltpu.async_copy(h2_ref.at[idx_s], gbuf, sem).wait()

        def mbody(e, carry2):
            wv = wrow[e, :]
            for j in range(4):
                gbuf[e, pl.ds(j * 16, 16)] = gbuf[e, pl.ds(j * 16, 16)] * wv
            return carry2

        lax.fori_loop(0, CHUNK, mbody, 0)
        pltpu.sync_copy(gbuf, accum.at[idx_d], add=True)
        return carry

    lax.fori_loop(0, NCHUNK, chunk_body, 0)
    plsc.subcore_barrier()

    _flush_stripe(accum, gbuf, lambda r, n: num_ref.at[c, pl.ds(r, n)], row0)


def _sc2(enc_p, t2_p, h2_p):
    f = pl.kernel(
        _sc2_body,
        out_type=[
            jax.ShapeDtypeStruct((2, N_PAD, 64), jnp.float32),
        ],
        mesh=_mesh,
        compiler_params=pltpu.CompilerParams(use_tc_tiling_on_sc=False),
        scratch_types=[
            pltpu.VMEM((CHUNK,), jnp.int32),
            pltpu.VMEM((CHUNK,), jnp.int32),
            pltpu.VMEM((CHUNK,), jnp.int32),
            pltpu.VMEM((CHUNK, 64), jnp.float32),
            pltpu.VMEM((CHUNK, 16), jnp.float32),
            pltpu.VMEM((CHUNK, 16), jnp.float32),
            pltpu.VMEM((CHUNK, 16), jnp.float32),
            pltpu.VMEM((ZROWS, 64), jnp.float32),
            pltpu.VMEM_SHARED((N_PAD, 64), jnp.float32),
            pltpu.SemaphoreType.DMA,
        ],
    )
    return f(enc_p, t2_p, h2_p)


def _tc_c_body(num_ref, t2_ref, h2_ref, b2_ref, out_ref):
    t2 = t2_ref[...]
    s2 = t2[:, 0:1] + t2[:, 8:9]
    w1c = _leaky_exp(s2)
    num = num_ref[0] + num_ref[1]
    numf = num + w1c * h2_ref[...]
    o = numf[:, 0:40] / (numf[:, 40:41] + 1e-16) + b2_ref[...]
    m = jnp.max(o, axis=1, keepdims=True)
    z = jnp.exp(o - m)
    lse = jnp.log(jnp.sum(z, axis=1, keepdims=True))
    out_ref[...] = o - m - lse


def _tc_c(num2, t2, h2, b2):
    return pl.pallas_call(
        _tc_c_body,
        grid=(NBLK,),
        in_specs=[
            pl.BlockSpec((2, BN, 64), lambda i: (0, i, 0)),
            pl.BlockSpec((BN, 16), lambda i: (i, 0)),
            pl.BlockSpec((BN, 64), lambda i: (i, 0)),
            pl.BlockSpec((40,), lambda i: (0,)),
        ],
        out_specs=pl.BlockSpec((BN, 40), lambda i: (i, 0)),
        out_shape=jax.ShapeDtypeStruct((N, 40), jnp.float32),
    )(num2, t2, h2, b2)


def kernel(x, edge_index, W1, a_src1, a_dst1, b1, W2, a_src2, a_dst2, b2):
    ei = edge_index.astype(jnp.int32)
    src2d = ei[0].reshape(EROWS, CHUNK)
    dst2d = ei[1].reshape(EROWS, CHUNK)
    enc2d = _tc_d(src2d, dst2d)
    enc_p = enc2d.reshape(E_PAD)

    I8s = jnp.eye(8, 16, dtype=jnp.float32)
    I8d = jnp.eye(8, 16, k=8, dtype=jnp.float32)
    A1 = (a_src1[:, :, None] * I8s[:, None, :]
          + a_dst1[:, :, None] * I8d[:, None, :]).reshape(D1, 16)

    *h1g, t1 = _tc_a(x, W1, A1)

    num1, den1 = _sc1(enc_p, t1, h1g)

    W2p = jnp.concatenate([W2, jnp.zeros((D1, 24), jnp.float32)], axis=1)
    v2s = jnp.concatenate([a_src2[0], jnp.zeros((24,), jnp.float32)])
    v2d = jnp.concatenate([a_dst2[0], jnp.zeros((24,), jnp.float32)])
    A2 = jnp.concatenate([jnp.broadcast_to(v2s[:, None], (64, 8)),
                          jnp.broadcast_to(v2d[:, None], (64, 8))], axis=1)

    h2, t2 = _tc_b(num1, den1, t1, h1g, b1, W2p, A2)

    num2, = _sc2(enc_p, t2, h2)

    return _tc_c(num2, t2, h2, b2)


# speedup vs baseline: 1.3231x; 1.3231x over previous
import jax
import jax.numpy as jnp
from jax import lax
from jax.experimental import pallas as pl
from jax.experimental.pallas import tpu as pltpu
from jax.experimental.pallas import tpu_sc as plsc

N = 10000
E = 160000
F_IN = 256
D1 = 512
BN = 400
N_PAD = 10400
NBLK_PAD = N_PAD // BN
NBLK = N // BN
E_PAD = 163840
EPT = E_PAD // 32
CHUNK = 128
NCHUNK = EPT // CHUNK
STRIPE = N_PAD // 16
FULL_FLUSH = STRIPE // CHUNK
TAIL_FLUSH = STRIPE % CHUNK
ZROWS = 64
EROWS = E // CHUNK
EROWS_PAD = E_PAD // CHUNK
SENT = (N << 14) | N

_mesh = plsc.VectorSubcoreMesh(core_axis_name="c", subcore_axis_name="s")


def _leaky_exp(v):
    return jnp.exp(jnp.maximum(v, 0.2 * v))


def _tc_d_body(src_ref, dst_ref, enc_ref):
    enc = (src_ref[...] << 14) | dst_ref[...]
    pad = jnp.full((EROWS_PAD - EROWS, CHUNK), SENT, jnp.int32)
    enc_ref[...] = jnp.concatenate([enc, pad], axis=0)


def _tc_d(src2d, dst2d):
    return pl.pallas_call(
        _tc_d_body,
        out_shape=jax.ShapeDtypeStruct((EROWS_PAD, CHUNK), jnp.int32),
    )(src2d, dst2d)


def _tc_a_body(x_ref, w1_ref, a1_ref, *out_refs):
    h = jnp.dot(x_ref[...], w1_ref[...], preferred_element_type=jnp.float32)
    for g in range(8):
        out_refs[g][...] = h[:, g * 64:(g + 1) * 64]
    out_refs[8][...] = jnp.dot(h, a1_ref[...],
                               preferred_element_type=jnp.float32)


def _tc_a(x, W1, A1):
    return pl.pallas_call(
        _tc_a_body,
        grid=(NBLK_PAD,),
        in_specs=[
            pl.BlockSpec((BN, F_IN), lambda i: (jnp.minimum(i, NBLK - 1), 0)),
            pl.BlockSpec((F_IN, D1), lambda i: (0, 0)),
            pl.BlockSpec((D1, 16), lambda i: (0, 0)),
        ],
        out_specs=[pl.BlockSpec((BN, 64), lambda i: (i, 0))] * 8
        + [pl.BlockSpec((BN, 16), lambda i: (i, 0))],
        out_shape=[jax.ShapeDtypeStruct((N_PAD, 64), jnp.float32)] * 8
        + [jax.ShapeDtypeStruct((N_PAD, 16), jnp.float32)],
    )(x, W1, A1)


def _zero_stripe(zb, acc, row0):
    nfull = STRIPE // ZROWS
    for t in range(nfull):
        pltpu.sync_copy(zb, acc.at[pl.ds(row0 + t * ZROWS, ZROWS)])
    tail = STRIPE % ZROWS
    if tail:
        pltpu.sync_copy(zb.at[pl.ds(0, tail)],
                        acc.at[pl.ds(row0 + nfull * ZROWS, tail)])


def _flush_stripe(acc, buf, out_at, row0):
    for t in range(FULL_FLUSH):
        r = row0 + t * CHUNK
        pltpu.sync_copy(acc.at[pl.ds(r, CHUNK)], buf)
        pltpu.sync_copy(buf, out_at(r, CHUNK))
    r = row0 + FULL_FLUSH * CHUNK
    pltpu.sync_copy(acc.at[pl.ds(r, TAIL_FLUSH)], buf.at[pl.ds(0, TAIL_FLUSH)])
    pltpu.sync_copy(buf.at[pl.ds(0, TAIL_FLUSH)], out_at(r, TAIL_FLUSH))


def _sc1_body(enc_ref, t1_ref, h0_ref, h1_ref, h2_ref, h3_ref, h4_ref,
              h5_ref, h6_ref, h7_ref, num_ref, den_ref,
              ebuf, idx_s, idx_d, wstore, gbuf, tsb, tdb, zb, zbn, accum,
              dacc, sem, semd, semh):
    c = lax.axis_index("c")
    s = lax.axis_index("s")
    tid = c * 16 + s
    base = tid * EPT
    row0 = s * STRIPE
    shift8 = (lax.iota(jnp.int32, 16) + 8) % 16

    def zrow(i, carry):
        for j in range(4):
            zb[i, pl.ds(j * 16, 16)] = jnp.zeros((16,), jnp.float32)
        return carry

    lax.fori_loop(0, ZROWS, zrow, 0)

    htabs = [h0_ref, h1_ref, h2_ref, h3_ref, h4_ref, h5_ref, h6_ref, h7_ref]
    for cg in range(8):
        _zero_stripe(zb, accum, row0)
        if cg == 0:
            _zero_stripe(zbn, dacc, row0)
        plsc.subcore_barrier()

        def chunk_body(k, carry, cg=cg):
            off = pl.multiple_of(base + k * CHUNK, CHUNK)
            pltpu.sync_copy(enc_ref.at[pl.ds(off, CHUNK)], ebuf)
            for i in range(CHUNK // 16):
                ev = ebuf[pl.ds(i * 16, 16)]
                idx_s[pl.ds(i * 16, 16)] = ev >> 14
                idx_d[pl.ds(i * 16, 16)] = ev & 16383
            dh = pltpu.async_copy(htabs[cg].at[idx_s], gbuf, semh)
            ds_ = pltpu.async_copy(t1_ref.at[idx_s], tsb, sem)
            dd = pltpu.async_copy(t1_ref.at[idx_d], tdb, semd)
            ds_.wait()
            dd.wait()

            def wbody(e, carry2):
                u = tsb[e, :]
                v = tdb[e, :]
                wstore[e, :] = _leaky_exp(u + v[shift8])
                return carry2

            lax.fori_loop(0, CHUNK, wbody, 0)
            if cg == 0:
                pltpu.sync_copy(wstore, dacc.at[idx_d], add=True)
            dh.wait()

            def mbody(e, carry2, cg=cg):
                w0 = wstore[e, :][cg]
                for j in range(4):
                    gbuf[e, pl.ds(j * 16, 16)] = gbuf[e, pl.ds(j * 16, 16)] * w0
                return carry2

            lax.fori_loop(0, CHUNK, mbody, 0)
            pltpu.sync_copy(gbuf, accum.at[idx_d], add=True)
            return carry

        lax.fori_loop(0, NCHUNK, chunk_body, 0)
        plsc.subcore_barrier()

        _flush_stripe(accum, gbuf,
                      lambda r, n, cg=cg: num_ref.at[c, cg, pl.ds(r, n)], row0)
        if cg == 0:
            _flush_stripe(dacc, tsb,
                          lambda r, n: den_ref.at[c, pl.ds(r, n)], row0)


def _sc1(enc_p, t1_p, htabs):
    f = pl.kernel(
        _sc1_body,
        out_type=[
            jax.ShapeDtypeStruct((2, 8, N_PAD, 64), jnp.float32),
            jax.ShapeDtypeStruct((2, N_PAD, 16), jnp.float32),
        ],
        mesh=_mesh,
        compiler_params=pltpu.CompilerParams(use_tc_tiling_on_sc=False),
        scratch_types=[
            pltpu.VMEM((CHUNK,), jnp.int32),
            pltpu.VMEM((CHUNK,), jnp.int32),
            pltpu.VMEM((CHUNK,), jnp.int32),
            pltpu.VMEM((CHUNK, 16), jnp.float32),
            pltpu.VMEM((CHUNK, 64), jnp.float32),
            pltpu.VMEM((CHUNK, 16), jnp.float32),
            pltpu.VMEM((CHUNK, 16), jnp.float32),
            pltpu.VMEM((ZROWS, 64), jnp.float32),
            pltpu.VMEM((ZROWS, 16), jnp.float32),
            pltpu.VMEM_SHARED((N_PAD, 64), jnp.float32),
            pltpu.VMEM_SHARED((N_PAD, 16), jnp.float32),
            pltpu.SemaphoreType.DMA,
            pltpu.SemaphoreType.DMA,
            pltpu.SemaphoreType.DMA,
        ],
    )
    return f(enc_p, t1_p, *htabs)


def _tc_b_body(num_ref, den_ref, t1_ref, h1g0, h1g1, h1g2, h1g3, h1g4,
               h1g5, h1g6, h1g7, b1_ref, w2_ref, a2_ref, h2_ref, t2_ref):
    hg = [h1g0, h1g1, h1g2, h1g3, h1g4, h1g5, h1g6, h1g7]
    nsum = num_ref[0] + num_ref[1]
    ncat = jnp.concatenate([nsum[g] for g in range(8)], axis=-1)
    hcat = jnp.concatenate([r[...] for r in hg], axis=-1)
    t1 = t1_ref[...]
    s1 = t1[:, 0:8] + t1[:, 8:16]
    w8 = _leaky_exp(s1)
    wc = jnp.concatenate(
        [jnp.broadcast_to(w8[:, h:h + 1], (BN, 64)) for h in range(8)],
        axis=-1)
    d8 = den_ref[0] + den_ref[1]
    dtot = d8[:, 0:8] + w8
    dc = jnp.concatenate(
        [jnp.broadcast_to(dtot[:, h:h + 1], (BN, 64)) for h in range(8)],
        axis=-1)
    numf = ncat + wc * hcat
    out1 = numf / (dc + 1e-16) + b1_ref[...]
    act = jnp.where(out1 > 0, out1,
                    jnp.exp(jnp.minimum(out1, 0.0)) - 1.0)
    h2 = jnp.dot(act, w2_ref[...], preferred_element_type=jnp.float32)
    ones = jnp.ones((BN, 8), jnp.float32)
    zeros = jnp.zeros((BN, 16), jnp.float32)
    h2_ref[...] = jnp.concatenate([h2[:, 0:40], ones, zeros], axis=-1)
    t2_ref[...] = jnp.dot(h2, a2_ref[...], preferred_element_type=jnp.float32)


def _tc_b(num1, den1, t1, h1g, b1, W2p, A2):
    return pl.pallas_call(
        _tc_b_body,
        grid=(NBLK_PAD,),
        in_specs=[
            pl.BlockSpec((2, 8, BN, 64), lambda i: (0, 0, i, 0)),
            pl.BlockSpec((2, BN, 16), lambda i: (0, i, 0)),
            pl.BlockSpec((BN, 16), lambda i: (i, 0)),
        ] + [pl.BlockSpec((BN, 64), lambda i: (i, 0))] * 8 + [
            pl.BlockSpec((D1,), lambda i: (0,)),
            pl.BlockSpec((D1, 64), lambda i: (0, 0)),
            pl.BlockSpec((64, 16), lambda i: (0, 0)),
        ],
        out_specs=[
            pl.BlockSpec((BN, 64), lambda i: (i, 0)),
            pl.BlockSpec((BN, 16), lambda i: (i, 0)),
        ],
        out_shape=[
            jax.ShapeDtypeStruct((N_PAD, 64), jnp.float32),
            jax.ShapeDtypeStruct((N_PAD, 16), jnp.float32),
        ],
    )(num1, den1, t1, *h1g, b1, W2p, A2)


def _sc2_body(enc_ref, t2_ref, h2_ref, num_ref,
              ebuf, idx_s, idx_d, gbuf, wrow, tsb, tdb, zb, accum, sem, semd,
              semh):
    c = lax.axis_index("c")
    s = lax.axis_index("s")
    tid = c * 16 + s
    base = tid * EPT
    row0 = s * STRIPE
    shift8 = (lax.iota(jnp.int32, 16) + 8) % 16
    low8 = lax.iota(jnp.int32, 16) % 8

    def zrow(i, carry):
        for j in range(4):
            zb[i, pl.ds(j * 16, 16)] = jnp.zeros((16,), jnp.float32)
        return carry

    lax.fori_loop(0, ZROWS, zrow, 0)

    _zero_stripe(zb, accum, row0)
    plsc.subcore_barrier()

    def chunk_body(k, carry):
        off = pl.multiple_of(base + k * CHUNK, CHUNK)
        pltpu.sync_copy(enc_ref.at[pl.ds(off, CHUNK)], ebuf)
        for i in range(CHUNK // 16):
            ev = ebuf[pl.ds(i * 16, 16)]
            idx_s[pl.ds(i * 16, 16)] = ev >> 14
            idx_d[pl.ds(i * 16, 16)] = ev & 16383
        dh = pltpu.async_copy(h2_ref.at[idx_s], gbuf, semh)
        ds_ = pltpu.async_copy(t2_ref.at[idx_s], tsb, sem)
        dd = pltpu.async_copy(t2_ref.at[idx_d], tdb, semd)
        ds_.wait()
        dd.wait()

        def wbody(e, carry2):
            u = tsb[e, :]
            v = tdb[e, :]
            w = _leaky_exp(u + v[shift8])
            wrow[e, :] = w[low8]
            return carry2

        lax.fori_loop(0, CHUNK, wbody, 0)
        dh.wait()

        def mbody(e, carry2):
            wv = wrow[e, :]
            for j in range(4):
                gbuf[e, pl.ds(j * 16, 16)] = gbuf[e, pl.ds(j * 16, 16)] * wv
            return carry2

        lax.fori_loop(0, CHUNK, mbody, 0)
        pltpu.sync_copy(gbuf, accum.at[idx_d], add=True)
        return carry

    lax.fori_loop(0, NCHUNK, chunk_body, 0)
    plsc.subcore_barrier()

    _flush_stripe(accum, gbuf, lambda r, n: num_ref.at[c, pl.ds(r, n)], row0)


def _sc2(enc_p, t2_p, h2_p):
    f = pl.kernel(
        _sc2_body,
        out_type=[
            jax.ShapeDtypeStruct((2, N_PAD, 64), jnp.float32),
        ],
        mesh=_mesh,
        compiler_params=pltpu.CompilerParams(use_tc_tiling_on_sc=False),
        scratch_types=[
            pltpu.VMEM((CHUNK,), jnp.int32),
            pltpu.VMEM((CHUNK,), jnp.int32),
            pltpu.VMEM((CHUNK,), jnp.int32),
            pltpu.VMEM((CHUNK, 64), jnp.float32),
            pltpu.VMEM((CHUNK, 16), jnp.float32),
            pltpu.VMEM((CHUNK, 16), jnp.float32),
            pltpu.VMEM((CHUNK, 16), jnp.float32),
            pltpu.VMEM((ZROWS, 64), jnp.float32),
            pltpu.VMEM_SHARED((N_PAD, 64), jnp.float32),
            pltpu.SemaphoreType.DMA,
            pltpu.SemaphoreType.DMA,
            pltpu.SemaphoreType.DMA,
        ],
    )
    return f(enc_p, t2_p, h2_p)


def _tc_c_body(num_ref, t2_ref, h2_ref, b2_ref, out_ref):
    t2 = t2_ref[...]
    s2 = t2[:, 0:1] + t2[:, 8:9]
    w1c = _leaky_exp(s2)
    num = num_ref[0] + num_ref[1]
    numf = num + w1c * h2_ref[...]
    o = numf[:, 0:40] / (numf[:, 40:41] + 1e-16) + b2_ref[...]
    m = jnp.max(o, axis=1, keepdims=True)
    z = jnp.exp(o - m)
    lse = jnp.log(jnp.sum(z, axis=1, keepdims=True))
    out_ref[...] = o - m - lse


def _tc_c(num2, t2, h2, b2):
    return pl.pallas_call(
        _tc_c_body,
        grid=(NBLK,),
        in_specs=[
            pl.BlockSpec((2, BN, 64), lambda i: (0, i, 0)),
            pl.BlockSpec((BN, 16), lambda i: (i, 0)),
            pl.BlockSpec((BN, 64), lambda i: (i, 0)),
            pl.BlockSpec((40,), lambda i: (0,)),
        ],
        out_specs=pl.BlockSpec((BN, 40), lambda i: (i, 0)),
        out_shape=jax.ShapeDtypeStruct((N, 40), jnp.float32),
    )(num2, t2, h2, b2)


def kernel(x, edge_index, W1, a_src1, a_dst1, b1, W2, a_src2, a_dst2, b2):
    ei = edge_index.astype(jnp.int32)
    src2d = ei[0].reshape(EROWS, CHUNK)
    dst2d = ei[1].reshape(EROWS, CHUNK)
    enc2d = _tc_d(src2d, dst2d)
    enc_p = enc2d.reshape(E_PAD)

    I8s = jnp.eye(8, 16, dtype=jnp.float32)
    I8d = jnp.eye(8, 16, k=8, dtype=jnp.float32)
    A1 = (a_src1[:, :, None] * I8s[:, None, :]
          + a_dst1[:, :, None] * I8d[:, None, :]).reshape(D1, 16)

    *h1g, t1 = _tc_a(x, W1, A1)

    num1, den1 = _sc1(enc_p, t1, h1g)

    W2p = jnp.concatenate([W2, jnp.zeros((D1, 24), jnp.float32)], axis=1)
    v2s = jnp.concatenate([a_src2[0], jnp.zeros((24,), jnp.float32)])
    v2d = jnp.concatenate([a_dst2[0], jnp.zeros((24,), jnp.float32)])
    A2 = jnp.concatenate([jnp.broadcast_to(v2s[:, None], (64, 8)),
                          jnp.broadcast_to(v2d[:, None], (64, 8))], axis=1)

    h2, t2 = _tc_b(num1, den1, t1, h1g, b1, W2p, A2)

    num2, = _sc2(enc_p, t2, h2)

    return _tc_c(num2, t2, h2, b2)


# speedup vs baseline: 1.5528x; 1.1736x over previous
import jax
import jax.numpy as jnp
from jax import lax
from jax.experimental import pallas as pl
from jax.experimental.pallas import tpu as pltpu
from jax.experimental.pallas import tpu_sc as plsc

N = 10000
E = 160000
F_IN = 256
D1 = 512
BN = 400
N_PAD = 10400
NBLK_PAD = N_PAD // BN
NBLK = N // BN
E_PAD = 163840
EPT = E_PAD // 32
CHUNK = 128
NCHUNK = EPT // CHUNK
STRIPE = N_PAD // 16
FULL_FLUSH = STRIPE // CHUNK
TAIL_FLUSH = STRIPE % CHUNK
ZROWS = 64
EROWS = E // CHUNK
EROWS_PAD = E_PAD // CHUNK
SENT = (N << 14) | N

_mesh = plsc.VectorSubcoreMesh(core_axis_name="c", subcore_axis_name="s")


def _leaky_exp(v):
    return jnp.exp(jnp.maximum(v, 0.2 * v))


def _tc_d_body(src_ref, dst_ref, enc_ref, z64_ref, z16_ref):
    enc = (src_ref[...] << 14) | dst_ref[...]
    pad = jnp.full((EROWS_PAD - EROWS, CHUNK), SENT, jnp.int32)
    enc_ref[...] = jnp.concatenate([enc, pad], axis=0)
    z64_ref[...] = jnp.zeros((ZROWS, 64), jnp.float32)
    z16_ref[...] = jnp.zeros((ZROWS, 16), jnp.float32)


def _tc_d(src2d, dst2d):
    return pl.pallas_call(
        _tc_d_body,
        out_shape=[
            jax.ShapeDtypeStruct((EROWS_PAD, CHUNK), jnp.int32),
            jax.ShapeDtypeStruct((ZROWS, 64), jnp.float32),
            jax.ShapeDtypeStruct((ZROWS, 16), jnp.float32),
        ],
    )(src2d, dst2d)


def _tc_a_body(x_ref, w1_ref, a1_ref, *out_refs):
    h = jnp.dot(x_ref[...], w1_ref[...], preferred_element_type=jnp.float32)
    for g in range(8):
        out_refs[g][...] = h[:, g * 64:(g + 1) * 64]
    out_refs[8][...] = jnp.dot(h, a1_ref[...],
                               preferred_element_type=jnp.float32)


def _tc_a(x, W1, A1):
    return pl.pallas_call(
        _tc_a_body,
        grid=(NBLK_PAD,),
        in_specs=[
            pl.BlockSpec((BN, F_IN), lambda i: (jnp.minimum(i, NBLK - 1), 0)),
            pl.BlockSpec((F_IN, D1), lambda i: (0, 0)),
            pl.BlockSpec((D1, 16), lambda i: (0, 0)),
        ],
        out_specs=[pl.BlockSpec((BN, 64), lambda i: (i, 0))] * 8
        + [pl.BlockSpec((BN, 16), lambda i: (i, 0))],
        out_shape=[jax.ShapeDtypeStruct((N_PAD, 64), jnp.float32)] * 8
        + [jax.ShapeDtypeStruct((N_PAD, 16), jnp.float32)],
    )(x, W1, A1)


def _zero_stripe(zb, acc, row0):
    nfull = STRIPE // ZROWS
    for t in range(nfull):
        pltpu.sync_copy(zb, acc.at[pl.ds(row0 + t * ZROWS, ZROWS)])
    tail = STRIPE % ZROWS
    if tail:
        pltpu.sync_copy(zb.at[pl.ds(0, tail)],
                        acc.at[pl.ds(row0 + nfull * ZROWS, tail)])


def _flush_stripe(acc, buf, out_at, row0):
    for t in range(FULL_FLUSH):
        r = row0 + t * CHUNK
        pltpu.sync_copy(acc.at[pl.ds(r, CHUNK)], buf)
        pltpu.sync_copy(buf, out_at(r, CHUNK))
    r = row0 + FULL_FLUSH * CHUNK
    pltpu.sync_copy(acc.at[pl.ds(r, TAIL_FLUSH)], buf.at[pl.ds(0, TAIL_FLUSH)])
    pltpu.sync_copy(buf.at[pl.ds(0, TAIL_FLUSH)], out_at(r, TAIL_FLUSH))


def _sc1_body(enc_ref, t1_ref, z64_ref, z16_ref, h0_ref, h1_ref, h2_ref,
              h3_ref, h4_ref, h5_ref, h6_ref, h7_ref, num_ref, den_ref,
              ebuf, ia_s, ia_d, ib_s, ib_d, wsa, wsb, ga, gb, tsa, tda, tsb,
              tdb, accum, dacc,
              sha, ssa, sda, shb, ssb, sdb):
    c = lax.axis_index("c")
    s = lax.axis_index("s")
    tid = c * 16 + s
    base = tid * EPT
    row0 = s * STRIPE
    shift8 = (lax.iota(jnp.int32, 16) + 8) % 16

    def decode(off, i_s, i_d):
        pltpu.sync_copy(enc_ref.at[pl.ds(off, CHUNK)], ebuf)
        for i in range(CHUNK // 16):
            ev = ebuf[pl.ds(i * 16, 16)]
            i_s[pl.ds(i * 16, 16)] = ev >> 14
            i_d[pl.ds(i * 16, 16)] = ev & 16383

    def launch(htab, i_s, i_d, g, ts, td, sh, ss, sd):
        pltpu.async_copy(htab.at[i_s], g, sh)
        pltpu.async_copy(t1_ref.at[i_s], ts, ss)
        pltpu.async_copy(t1_ref.at[i_d], td, sd)

    def wait_all(htab, i_s, i_d, g, ts, td, sh, ss, sd):
        pltpu.make_async_copy(t1_ref.at[i_s], ts, ss).wait()
        pltpu.make_async_copy(t1_ref.at[i_d], td, sd).wait()
        pltpu.make_async_copy(htab.at[i_s], g, sh).wait()

    def compute(cg, i_d, g, ts, td, wst):
        def wbody(e, carry2):
            u = ts[e, :]
            v = td[e, :]
            wst[e, :] = _leaky_exp(u + v[shift8])
            return carry2

        lax.fori_loop(0, CHUNK, wbody, 0)
        if cg == 0:
            pltpu.sync_copy(wst, dacc.at[i_d], add=True)

        def mbody(e, carry2, cg=cg):
            w0 = wst[e, :][cg]
            for j in range(4):
                g[e, pl.ds(j * 16, 16)] = g[e, pl.ds(j * 16, 16)] * w0
            return carry2

        lax.fori_loop(0, CHUNK, mbody, 0)
        pltpu.sync_copy(g, accum.at[i_d], add=True)

    htabs = [h0_ref, h1_ref, h2_ref, h3_ref, h4_ref, h5_ref, h6_ref, h7_ref]
    for cg in range(8):
        htab = htabs[cg]
        _zero_stripe(z64_ref, accum, row0)
        if cg == 0:
            _zero_stripe(z16_ref, dacc, row0)
        plsc.subcore_barrier()

        decode(pl.multiple_of(base, CHUNK), ia_s, ia_d)
        launch(htab, ia_s, ia_d, ga, tsa, tda, sha, ssa, sda)

        def pair(p, carry, cg=cg, htab=htab):
            off1 = pl.multiple_of(base + (2 * p + 1) * CHUNK, CHUNK)
            decode(off1, ib_s, ib_d)
            launch(htab, ib_s, ib_d, gb, tsb, tdb, shb, ssb, sdb)
            wait_all(htab, ia_s, ia_d, ga, tsa, tda, sha, ssa, sda)
            compute(cg, ia_d, ga, tsa, tda, wsa)
            nxt = 2 * p + 2

            @pl.when(nxt < NCHUNK)
            def _():
                offn = pl.multiple_of(base + nxt * CHUNK, CHUNK)
                decode(offn, ia_s, ia_d)
                launch(htab, ia_s, ia_d, ga, tsa, tda, sha, ssa, sda)

            wait_all(htab, ib_s, ib_d, gb, tsb, tdb, shb, ssb, sdb)
            compute(cg, ib_d, gb, tsb, tdb, wsb)
            return carry

        lax.fori_loop(0, NCHUNK // 2, pair, 0)
        plsc.subcore_barrier()

        _flush_stripe(accum, ga,
                      lambda r, n, cg=cg: num_ref.at[c, cg, pl.ds(r, n)], row0)
        if cg == 0:
            _flush_stripe(dacc, tsa,
                          lambda r, n: den_ref.at[c, pl.ds(r, n)], row0)


def _sc1(enc_p, t1_p, z64, z16, htabs):
    f = pl.kernel(
        _sc1_body,
        out_type=[
            jax.ShapeDtypeStruct((2, 8, N_PAD, 64), jnp.float32),
            jax.ShapeDtypeStruct((2, N_PAD, 16), jnp.float32),
        ],
        mesh=_mesh,
        compiler_params=pltpu.CompilerParams(use_tc_tiling_on_sc=False),
        scratch_types=[
            pltpu.VMEM((CHUNK,), jnp.int32),
            pltpu.VMEM((CHUNK,), jnp.int32),
            pltpu.VMEM((CHUNK,), jnp.int32),
            pltpu.VMEM((CHUNK,), jnp.int32),
            pltpu.VMEM((CHUNK,), jnp.int32),
            pltpu.VMEM((CHUNK, 16), jnp.float32),
            pltpu.VMEM((CHUNK, 16), jnp.float32),
            pltpu.VMEM((CHUNK, 64), jnp.float32),
            pltpu.VMEM((CHUNK, 64), jnp.float32),
            pltpu.VMEM((CHUNK, 16), jnp.float32),
            pltpu.VMEM((CHUNK, 16), jnp.float32),
            pltpu.VMEM((CHUNK, 16), jnp.float32),
            pltpu.VMEM((CHUNK, 16), jnp.float32),
            pltpu.VMEM_SHARED((N_PAD, 64), jnp.float32),
            pltpu.VMEM_SHARED((N_PAD, 16), jnp.float32),
            pltpu.SemaphoreType.DMA,
            pltpu.SemaphoreType.DMA,
            pltpu.SemaphoreType.DMA,
            pltpu.SemaphoreType.DMA,
            pltpu.SemaphoreType.DMA,
            pltpu.SemaphoreType.DMA,
        ],
    )
    return f(enc_p, t1_p, z64, z16, *htabs)


def _tc_b_body(num_ref, den_ref, t1_ref, h1g0, h1g1, h1g2, h1g3, h1g4,
               h1g5, h1g6, h1g7, b1_ref, w2_ref, a2_ref, h2_ref, t2_ref):
    hg = [h1g0, h1g1, h1g2, h1g3, h1g4, h1g5, h1g6, h1g7]
    nsum = num_ref[0] + num_ref[1]
    ncat = jnp.concatenate([nsum[g] for g in range(8)], axis=-1)
    hcat = jnp.concatenate([r[...] for r in hg], axis=-1)
    t1 = t1_ref[...]
    s1 = t1[:, 0:8] + t1[:, 8:16]
    w8 = _leaky_exp(s1)
    wc = jnp.concatenate(
        [jnp.broadcast_to(w8[:, h:h + 1], (BN, 64)) for h in range(8)],
        axis=-1)
    d8 = den_ref[0] + den_ref[1]
    dtot = d8[:, 0:8] + w8
    dc = jnp.concatenate(
        [jnp.broadcast_to(dtot[:, h:h + 1], (BN, 64)) for h in range(8)],
        axis=-1)
    numf = ncat + wc * hcat
    out1 = numf / (dc + 1e-16) + b1_ref[...]
    act = jnp.where(out1 > 0, out1,
                    jnp.exp(jnp.minimum(out1, 0.0)) - 1.0)
    h2 = jnp.dot(act, w2_ref[...], preferred_element_type=jnp.float32)
    ones = jnp.ones((BN, 8), jnp.float32)
    zeros = jnp.zeros((BN, 16), jnp.float32)
    h2_ref[...] = jnp.concatenate([h2[:, 0:40], ones, zeros], axis=-1)
    t2_ref[...] = jnp.dot(h2, a2_ref[...], preferred_element_type=jnp.float32)


def _tc_b(num1, den1, t1, h1g, b1, W2p, A2):
    return pl.pallas_call(
        _tc_b_body,
        grid=(NBLK_PAD,),
        in_specs=[
            pl.BlockSpec((2, 8, BN, 64), lambda i: (0, 0, i, 0)),
            pl.BlockSpec((2, BN, 16), lambda i: (0, i, 0)),
            pl.BlockSpec((BN, 16), lambda i: (i, 0)),
        ] + [pl.BlockSpec((BN, 64), lambda i: (i, 0))] * 8 + [
            pl.BlockSpec((D1,), lambda i: (0,)),
            pl.BlockSpec((D1, 64), lambda i: (0, 0)),
            pl.BlockSpec((64, 16), lambda i: (0, 0)),
        ],
        out_specs=[
            pl.BlockSpec((BN, 64), lambda i: (i, 0)),
            pl.BlockSpec((BN, 16), lambda i: (i, 0)),
        ],
        out_shape=[
            jax.ShapeDtypeStruct((N_PAD, 64), jnp.float32),
            jax.ShapeDtypeStruct((N_PAD, 16), jnp.float32),
        ],
    )(num1, den1, t1, *h1g, b1, W2p, A2)


def _sc2_body(enc_ref, t2_ref, h2_ref, z64_ref, num_ref,
              ebuf, ia_s, ia_d, ib_s, ib_d, wsa, wsb, ga, gb, tsa, tda, tsb,
              tdb, accum,
              sha, ssa, sda, shb, ssb, sdb):
    c = lax.axis_index("c")
    s = lax.axis_index("s")
    tid = c * 16 + s
    base = tid * EPT
    row0 = s * STRIPE
    shift8 = (lax.iota(jnp.int32, 16) + 8) % 16
    low8 = lax.iota(jnp.int32, 16) % 8

    _zero_stripe(z64_ref, accum, row0)
    plsc.subcore_barrier()

    def decode(off, i_s, i_d):
        pltpu.sync_copy(enc_ref.at[pl.ds(off, CHUNK)], ebuf)
        for i in range(CHUNK // 16):
            ev = ebuf[pl.ds(i * 16, 16)]
            i_s[pl.ds(i * 16, 16)] = ev >> 14
            i_d[pl.ds(i * 16, 16)] = ev & 16383

    def launch(i_s, i_d, g, ts, td, sh, ss, sd):
        pltpu.async_copy(h2_ref.at[i_s], g, sh)
        pltpu.async_copy(t2_ref.at[i_s], ts, ss)
        pltpu.async_copy(t2_ref.at[i_d], td, sd)

    def wait_all(i_s, i_d, g, ts, td, sh, ss, sd):
        pltpu.make_async_copy(t2_ref.at[i_s], ts, ss).wait()
        pltpu.make_async_copy(t2_ref.at[i_d], td, sd).wait()
        pltpu.make_async_copy(h2_ref.at[i_s], g, sh).wait()

    def compute(i_d, g, ts, td, wst):
        def wbody(e, carry2):
            u = ts[e, :]
            v = td[e, :]
            w = _leaky_exp(u + v[shift8])
            wst[e, :] = w[low8]
            return carry2

        lax.fori_loop(0, CHUNK, wbody, 0)

        def mbody(e, carry2):
            wv = wst[e, :]
            for j in range(4):
                g[e, pl.ds(j * 16, 16)] = g[e, pl.ds(j * 16, 16)] * wv
            return carry2

        lax.fori_loop(0, CHUNK, mbody, 0)
        pltpu.sync_copy(g, accum.at[i_d], add=True)

    decode(pl.multiple_of(base, CHUNK), ia_s, ia_d)
    launch(ia_s, ia_d, ga, tsa, tda, sha, ssa, sda)

    def pair(p, carry):
        off1 = pl.multiple_of(base + (2 * p + 1) * CHUNK, CHUNK)
        decode(off1, ib_s, ib_d)
        launch(ib_s, ib_d, gb, tsb, tdb, shb, ssb, sdb)
        wait_all(ia_s, ia_d, ga, tsa, tda, sha, ssa, sda)
        compute(ia_d, ga, tsa, tda, wsa)
        nxt = 2 * p + 2

        @pl.when(nxt < NCHUNK)
        def _():
            offn = pl.multiple_of(base + nxt * CHUNK, CHUNK)
            decode(offn, ia_s, ia_d)
            launch(ia_s, ia_d, ga, tsa, tda, sha, ssa, sda)

        wait_all(ib_s, ib_d, gb, tsb, tdb, shb, ssb, sdb)
        compute(ib_d, gb, tsb, tdb, wsb)
        return carry

    lax.fori_loop(0, NCHUNK // 2, pair, 0)
    plsc.subcore_barrier()

    _flush_stripe(accum, ga, lambda r, n: num_ref.at[c, pl.ds(r, n)], row0)


def _sc2(enc_p, t2_p, h2_p, z64):
    f = pl.kernel(
        _sc2_body,
        out_type=[
            jax.ShapeDtypeStruct((2, N_PAD, 64), jnp.float32),
        ],
        mesh=_mesh,
        compiler_params=pltpu.CompilerParams(use_tc_tiling_on_sc=False),
        scratch_types=[
            pltpu.VMEM((CHUNK,), jnp.int32),
            pltpu.VMEM((CHUNK,), jnp.int32),
            pltpu.VMEM((CHUNK,), jnp.int32),
            pltpu.VMEM((CHUNK,), jnp.int32),
            pltpu.VMEM((CHUNK,), jnp.int32),
            pltpu.VMEM((CHUNK, 16), jnp.float32),
            pltpu.VMEM((CHUNK, 16), jnp.float32),
            pltpu.VMEM((CHUNK, 64), jnp.float32),
            pltpu.VMEM((CHUNK, 64), jnp.float32),
            pltpu.VMEM((CHUNK, 16), jnp.float32),
            pltpu.VMEM((CHUNK, 16), jnp.float32),
            pltpu.VMEM((CHUNK, 16), jnp.float32),
            pltpu.VMEM((CHUNK, 16), jnp.float32),
            pltpu.VMEM_SHARED((N_PAD, 64), jnp.float32),
            pltpu.SemaphoreType.DMA,
            pltpu.SemaphoreType.DMA,
            pltpu.SemaphoreType.DMA,
            pltpu.SemaphoreType.DMA,
            pltpu.SemaphoreType.DMA,
            pltpu.SemaphoreType.DMA,
        ],
    )
    return f(enc_p, t2_p, h2_p, z64)


def _tc_c_body(num_ref, t2_ref, h2_ref, b2_ref, out_ref):
    t2 = t2_ref[...]
    s2 = t2[:, 0:1] + t2[:, 8:9]
    w1c = _leaky_exp(s2)
    num = num_ref[0] + num_ref[1]
    numf = num + w1c * h2_ref[...]
    o = numf[:, 0:40] / (numf[:, 40:41] + 1e-16) + b2_ref[...]
    m = jnp.max(o, axis=1, keepdims=True)
    z = jnp.exp(o - m)
    lse = jnp.log(jnp.sum(z, axis=1, keepdims=True))
    out_ref[...] = o - m - lse


def _tc_c(num2, t2, h2, b2):
    return pl.pallas_call(
        _tc_c_body,
        grid=(NBLK,),
        in_specs=[
            pl.BlockSpec((2, BN, 64), lambda i: (0, i, 0)),
            pl.BlockSpec((BN, 16), lambda i: (i, 0)),
            pl.BlockSpec((BN, 64), lambda i: (i, 0)),
            pl.BlockSpec((40,), lambda i: (0,)),
        ],
        out_specs=pl.BlockSpec((BN, 40), lambda i: (i, 0)),
        out_shape=jax.ShapeDtypeStruct((N, 40), jnp.float32),
    )(num2, t2, h2, b2)


def kernel(x, edge_index, W1, a_src1, a_dst1, b1, W2, a_src2, a_dst2, b2):
    ei = edge_index.astype(jnp.int32)
    src2d = ei[0].reshape(EROWS, CHUNK)
    dst2d = ei[1].reshape(EROWS, CHUNK)
    enc2d, z64, z16 = _tc_d(src2d, dst2d)
    enc_p = enc2d.reshape(E_PAD)

    I8s = jnp.eye(8, 16, dtype=jnp.float32)
    I8d = jnp.eye(8, 16, k=8, dtype=jnp.float32)
    A1 = (a_src1[:, :, None] * I8s[:, None, :]
          + a_dst1[:, :, None] * I8d[:, None, :]).reshape(D1, 16)

    *h1g, t1 = _tc_a(x, W1, A1)

    num1, den1 = _sc1(enc_p, t1, z64, z16, h1g)

    W2p = jnp.concatenate([W2, jnp.zeros((D1, 24), jnp.float32)], axis=1)
    v2s = jnp.concatenate([a_src2[0], jnp.zeros((24,), jnp.float32)])
    v2d = jnp.concatenate([a_dst2[0], jnp.zeros((24,), jnp.float32)])
    A2 = jnp.concatenate([jnp.broadcast_to(v2s[:, None], (64, 8)),
                          jnp.broadcast_to(v2d[:, None], (64, 8))], axis=1)

    h2, t2 = _tc_b(num1, den1, t1, h1g, b1, W2p, A2)

    num2, = _sc2(enc_p, t2, h2, z64)

    return _tc_c(num2, t2, h2, b2)
